# baseline (speedup 1.0000x reference)
import functools

import jax
import jax.numpy as jnp
from jax import lax
from jax.experimental import pallas as pl
from jax.experimental.pallas import tpu as pltpu
from jax.experimental.pallas import tpu_sc as plsc

N_EMBED = 8192
E_DIM = 256
BETA = 0.25
N_TOKENS = 8192

TM = 256
TN = 2048
NI = N_TOKENS // TM
NJ = N_EMBED // TN


def _argmin_body(z_ref, e_ref, zn2_ref, en2_ref, idx_ref, loss_ref, rep_sc, arg_sc, fmin_sc):
    i = pl.program_id(0)
    j = pl.program_id(1)

    @pl.when(j == 0)
    def _init():
        rep_sc[...] = jnp.full((TM, 1), jnp.inf, dtype=jnp.float32)
        arg_sc[...] = jnp.zeros((TM, 1), dtype=jnp.int32)
        fmin_sc[...] = jnp.full((TM, 1), jnp.inf, dtype=jnp.float32)

    z = z_ref[...]
    e = e_ref[...]
    zn2 = zn2_ref[...]
    en2 = en2_ref[...]
    d = lax.dot_general(
        z, e, (((1,), (1,)), ((), ())),
        preferred_element_type=jnp.float32,
    )
    s = (zn2 + en2) - 2.0 * d

    bmin = jnp.min(s, axis=1, keepdims=True)
    iota = lax.broadcasted_iota(jnp.int32, (TM, TN), 1)
    barg = jnp.min(
        jnp.where(s == bmin, iota, jnp.int32(2**31 - 1)), axis=1, keepdims=True
    ) + j * TN
    bminq = bmin.astype(jnp.bfloat16).astype(jnp.float32)
    better = bmin < rep_sc[...]
    arg_sc[...] = jnp.where(better, barg, arg_sc[...])
    rep_sc[...] = jnp.where(better, bminq, rep_sc[...])
    fmin_sc[...] = jnp.minimum(fmin_sc[...], bmin)

    @pl.when(j == NJ - 1)
    def _finish():
        idx_ref[...] = arg_sc[...]
        part = jnp.sum(fmin_sc[...])

        @pl.when(i == 0)
        def _zero():
            loss_ref[...] = jnp.zeros((1, 1), dtype=jnp.float32)

        loss_ref[...] = loss_ref[...] + part.reshape(1, 1)


def _argmin_call(zbf, ebf, zn2, en2row, interpret=False):
    return pl.pallas_call(
        _argmin_body,
        grid=(NI, NJ),
        in_specs=[
            pl.BlockSpec((TM, E_DIM), lambda i, j: (i, 0)),
            pl.BlockSpec((TN, E_DIM), lambda i, j: (j, 0)),
            pl.BlockSpec((TM, 1), lambda i, j: (i, 0)),
            pl.BlockSpec((1, TN), lambda i, j: (0, j)),
        ],
        out_specs=[
            pl.BlockSpec((TM, 1), lambda i, j: (i, 0)),
            pl.BlockSpec((1, 1), lambda i, j: (0, 0)),
        ],
        out_shape=[
            jax.ShapeDtypeStruct((N_TOKENS, 1), jnp.int32),
            jax.ShapeDtypeStruct((1, 1), jnp.float32),
        ],
        scratch_shapes=[
            pltpu.VMEM((TM, 1), jnp.float32),
            pltpu.VMEM((TM, 1), jnp.int32),
            pltpu.VMEM((TM, 1), jnp.float32),
        ],
        compiler_params=pltpu.CompilerParams(
            dimension_semantics=("arbitrary", "arbitrary"),
        ),
        interpret=interpret,
    )(zbf, ebf, zn2, en2row)


_NC = 2
_NS = 16
_NW = _NC * _NS
_B_PER_W = N_TOKENS // _NW


@functools.cache
def _build_sc_gather():
    @functools.partial(
        pl.kernel,
        mesh=plsc.VectorSubcoreMesh(core_axis_name="c", subcore_axis_name="s"),
        out_type=jax.ShapeDtypeStruct((N_TOKENS, E_DIM), jnp.float32),
        scratch_types=[
            pltpu.VMEM((_B_PER_W,), jnp.int32),
            pltpu.VMEM((_B_PER_W, E_DIM), jnp.float32),
            pltpu.SemaphoreType.DMA,
        ],
    )
    def _sc_gather(emb_hbm, idx_hbm, out_hbm, idx_v, rows_v, sem):
        wid = lax.axis_index("s") * _NC + lax.axis_index("c")
        base = wid * _B_PER_W
        pltpu.sync_copy(idx_hbm.at[pl.ds(base, _B_PER_W)], idx_v)
        pltpu.async_copy(emb_hbm.at[idx_v], rows_v, sem).wait()
        pltpu.sync_copy(rows_v, out_hbm.at[pl.ds(base, _B_PER_W)])

    return _sc_gather


def kernel(z, emb_weight, is_training):
    zf = z.reshape(-1, E_DIM)
    zn2 = jnp.sum(zf ** 2, axis=1, keepdims=True)
    en2row = jnp.sum(emb_weight ** 2, axis=1).reshape(1, N_EMBED)
    zbf = zf.astype(jnp.bfloat16)
    ebf = emb_weight.astype(jnp.bfloat16)
    idx2d, losssum = _argmin_call(zbf, ebf, zn2, en2row)
    idx = idx2d.reshape(-1)
    z_q = _build_sc_gather()(emb_weight, idx).reshape(z.shape)
    vq_loss = losssum[0, 0] / jnp.float32(N_TOKENS * E_DIM)
    commitment_loss = jnp.float32(BETA) * vq_loss
    return (z_q, vq_loss, commitment_loss, idx)


# speedup vs baseline: 1.4501x; 1.4501x over previous
import functools

import jax
import jax.numpy as jnp
from jax import lax
from jax.experimental import pallas as pl
from jax.experimental.pallas import tpu as pltpu
from jax.experimental.pallas import tpu_sc as plsc

N_EMBED = 8192
E_DIM = 256
BETA = 0.25
N_TOKENS = 8192

TM = 2048
TN = 2048
NI = N_TOKENS // TM
NJ = N_EMBED // TN


def _argmin_body(z_ref, e_ref, zn2_ref, en2_ref, idx_ref, loss_ref, rep_sc, arg_sc, fmin_sc):
    i = pl.program_id(0)
    j = pl.program_id(1)

    @pl.when(j == 0)
    def _init():
        rep_sc[...] = jnp.full((TM, 1), jnp.inf, dtype=jnp.float32)
        arg_sc[...] = jnp.zeros((TM, 1), dtype=jnp.int32)
        fmin_sc[...] = jnp.full((TM, 1), jnp.inf, dtype=jnp.float32)

    z = z_ref[...]
    e = e_ref[...]
    zn2 = zn2_ref[...]
    en2 = en2_ref[...]
    d = lax.dot_general(
        z, e, (((1,), (1,)), ((), ())),
        preferred_element_type=jnp.float32,
    )
    s = (zn2 + en2) - 2.0 * d

    bmin = jnp.min(s, axis=1, keepdims=True)
    iota = lax.broadcasted_iota(jnp.int32, (TM, TN), 1)
    barg = jnp.min(
        jnp.where(s == bmin, iota, jnp.int32(2**31 - 1)), axis=1, keepdims=True
    ) + j * TN
    bminq = bmin.astype(jnp.bfloat16).astype(jnp.float32)
    better = bmin < rep_sc[...]
    arg_sc[...] = jnp.where(better, barg, arg_sc[...])
    rep_sc[...] = jnp.where(better, bminq, rep_sc[...])
    fmin_sc[...] = jnp.minimum(fmin_sc[...], bmin)

    @pl.when(j == NJ - 1)
    def _finish():
        idx_ref[...] = arg_sc[...]
        part = jnp.sum(fmin_sc[...])

        @pl.when(i == 0)
        def _zero():
            loss_ref[...] = jnp.zeros((1, 1), dtype=jnp.float32)

        loss_ref[...] = loss_ref[...] + part.reshape(1, 1)


def _argmin_call(zbf, ebf, zn2, en2row, interpret=False):
    return pl.pallas_call(
        _argmin_body,
        grid=(NI, NJ),
        in_specs=[
            pl.BlockSpec((TM, E_DIM), lambda i, j: (i, 0)),
            pl.BlockSpec((TN, E_DIM), lambda i, j: (j, 0)),
            pl.BlockSpec((TM, 1), lambda i, j: (i, 0)),
            pl.BlockSpec((1, TN), lambda i, j: (0, j)),
        ],
        out_specs=[
            pl.BlockSpec((TM, 1), lambda i, j: (i, 0)),
            pl.BlockSpec((1, 1), lambda i, j: (0, 0)),
        ],
        out_shape=[
            jax.ShapeDtypeStruct((N_TOKENS, 1), jnp.int32),
            jax.ShapeDtypeStruct((1, 1), jnp.float32),
        ],
        scratch_shapes=[
            pltpu.VMEM((TM, 1), jnp.float32),
            pltpu.VMEM((TM, 1), jnp.int32),
            pltpu.VMEM((TM, 1), jnp.float32),
        ],
        compiler_params=pltpu.CompilerParams(
            dimension_semantics=("arbitrary", "arbitrary"),
        ),
        interpret=interpret,
    )(zbf, ebf, zn2, en2row)


_NC = 2
_NS = 16
_NW = _NC * _NS
_B_PER_W = N_TOKENS // _NW


@functools.cache
def _build_sc_gather():
    @functools.partial(
        pl.kernel,
        mesh=plsc.VectorSubcoreMesh(core_axis_name="c", subcore_axis_name="s"),
        out_type=jax.ShapeDtypeStruct((N_TOKENS, E_DIM), jnp.float32),
        scratch_types=[
            pltpu.VMEM((_B_PER_W,), jnp.int32),
            pltpu.VMEM((_B_PER_W, E_DIM), jnp.float32),
            pltpu.SemaphoreType.DMA,
        ],
    )
    def _sc_gather(emb_hbm, idx_hbm, out_hbm, idx_v, rows_v, sem):
        wid = lax.axis_index("s") * _NC + lax.axis_index("c")
        base = wid * _B_PER_W
        pltpu.sync_copy(idx_hbm.at[pl.ds(base, _B_PER_W)], idx_v)
        pltpu.async_copy(emb_hbm.at[idx_v], rows_v, sem).wait()
        pltpu.sync_copy(rows_v, out_hbm.at[pl.ds(base, _B_PER_W)])

    return _sc_gather


def kernel(z, emb_weight, is_training):
    zf = z.reshape(-1, E_DIM)
    zn2 = jnp.sum(zf ** 2, axis=1, keepdims=True)
    en2row = jnp.sum(emb_weight ** 2, axis=1).reshape(1, N_EMBED)
    zbf = zf.astype(jnp.bfloat16)
    ebf = emb_weight.astype(jnp.bfloat16)
    idx2d, losssum = _argmin_call(zbf, ebf, zn2, en2row)
    idx = idx2d.reshape(-1)
    z_q = _build_sc_gather()(emb_weight, idx).reshape(z.shape)
    vq_loss = losssum[0, 0] / jnp.float32(N_TOKENS * E_DIM)
    commitment_loss = jnp.float32(BETA) * vq_loss
    return (z_q, vq_loss, commitment_loss, idx)
